# baseline (device time: 26227 ns/iter reference)
import jax
import jax.numpy as jnp
from jax import lax
from jax.experimental import pallas as pl
from jax.experimental.pallas import tpu as pltpu

N_DEV = 16
CHUNK = 16


def kernel(x, Wq, Wo, K_ext, V_ext):
    B, Sq, D = x.shape
    H_loc = Wq.shape[1]
    Dh = K_ext.shape[-1]
    H = H_loc // Dh
    Dout = Wo.shape[1]
    CPB = N_DEV // B

    def body(x_ref, wq_ref, wo_ref, k_ref, v_ref, out_ref,
             part_ref, stage_ref, obuf_ref, rs_recv, ag_send, ag_recv,
             rs_send_sems, rs_recv_sems, ag_send_sems, ag_recv_sems):
        my = lax.axis_index("i")

        barrier = pltpu.get_barrier_semaphore()
        for o in range(1, N_DEV):
            pl.semaphore_signal(barrier, inc=1, device_id=(my ^ o,),
                                device_id_type=pl.DeviceIdType.MESH)
        pl.semaphore_wait(barrier, N_DEV - 1)

        rs = []
        for o in range(1, N_DEV):
            peer = my ^ o
            rs.append(pltpu.make_async_remote_copy(
                src_ref=stage_ref.at[peer],
                dst_ref=rs_recv.at[o],
                send_sem=rs_send_sems.at[o],
                recv_sem=rs_recv_sems.at[o],
                device_id=(peer,),
                device_id_type=pl.DeviceIdType.MESH,
            ))

        wq = wq_ref[...].astype(jnp.bfloat16)
        wo = wo_ref[...].astype(jnp.bfloat16)
        x2d = x_ref[...].reshape(B * Sq, D).astype(jnp.bfloat16)
        q = jnp.dot(x2d, wq, preferred_element_type=jnp.float32)
        for b in range(B):
            for h in range(H):
                qh = (q[b * Sq:(b + 1) * Sq, h * Dh:(h + 1) * Dh]
                      * 0.125).astype(jnp.bfloat16)
                kh = k_ref[b, :, h, :].astype(jnp.bfloat16)
                vh = v_ref[b, :, h, :].astype(jnp.bfloat16)
                s = jnp.dot(qh, kh.T, preferred_element_type=jnp.float32)
                m = jnp.max(s, axis=-1, keepdims=True)
                p = jnp.exp(s - m)
                l = jnp.sum(p, axis=-1, keepdims=True)
                o_h = jnp.dot(p.astype(jnp.bfloat16), vh,
                              preferred_element_type=jnp.float32) / l
                obuf_ref[b * Sq:(b + 1) * Sq,
                         h * Dh:(h + 1) * Dh] = o_h.astype(jnp.bfloat16)
            accb = jnp.dot(obuf_ref[b * Sq:(b + 1) * Sq, :], wo,
                           preferred_element_type=jnp.float32)
            part_ref[b * CPB:(b + 1) * CPB] = accb.reshape(CPB, CHUNK, Dout)
            stage_ref[b * CPB:(b + 1) * CPB] = (
                accb.astype(jnp.bfloat16).reshape(CPB, CHUNK, Dout))
            for o in range(1, N_DEV):
                peer = my ^ o
                rdma = rs[o - 1]

                @pl.when((peer // CPB) == b)
                def _(rdma=rdma):
                    rdma.start()

        red = part_ref[my]
        for o in range(1, N_DEV):
            rs[o - 1].wait_recv()
            red = red + rs_recv[o].astype(jnp.float32)

        ag_send[...] = red.astype(jnp.bfloat16)
        ag = []
        for o in range(1, N_DEV):
            rdma = pltpu.make_async_remote_copy(
                src_ref=ag_send,
                dst_ref=ag_recv.at[o],
                send_sem=ag_send_sems.at[o],
                recv_sem=ag_recv_sems.at[o],
                device_id=(my ^ o,),
                device_id_type=pl.DeviceIdType.MESH,
            )
            rdma.start()
            ag.append(rdma)

        out_ref[my] = red
        for o in range(1, N_DEV):
            ag[o - 1].wait_recv()
            out_ref[my ^ o] = ag_recv[o].astype(jnp.float32)

        for o in range(1, N_DEV):
            rs[o - 1].wait_send()
            ag[o - 1].wait_send()

    out = pl.pallas_call(
        body,
        out_shape=jax.ShapeDtypeStruct((N_DEV, CHUNK, Dout), jnp.float32),
        in_specs=[pl.BlockSpec(memory_space=pltpu.VMEM)] * 5,
        out_specs=pl.BlockSpec(memory_space=pltpu.VMEM),
        scratch_shapes=[
            pltpu.VMEM((N_DEV, CHUNK, Dout), jnp.float32),
            pltpu.VMEM((N_DEV, CHUNK, Dout), jnp.bfloat16),
            pltpu.VMEM((B * Sq, H * Dh), jnp.bfloat16),
            pltpu.VMEM((N_DEV, CHUNK, Dout), jnp.bfloat16),
            pltpu.VMEM((CHUNK, Dout), jnp.bfloat16),
            pltpu.VMEM((N_DEV, CHUNK, Dout), jnp.bfloat16),
            pltpu.SemaphoreType.DMA((N_DEV,)),
            pltpu.SemaphoreType.DMA((N_DEV,)),
            pltpu.SemaphoreType.DMA((N_DEV,)),
            pltpu.SemaphoreType.DMA((N_DEV,)),
        ],
        compiler_params=pltpu.CompilerParams(collective_id=0),
    )(x, Wq, Wo, K_ext, V_ext)
    return out.reshape(B, Sq, Dout)
